# baseline (speedup 1.0000x reference)
import functools

import jax
import jax.numpy as jnp
from jax import lax
from jax.experimental import pallas as pl
from jax.experimental.pallas import tpu as pltpu
from jax.experimental.pallas import tpu_sc as plsc

_B = 16384
_E = 64
_NC = 2
_NS = 16
_NW = _NC * _NS
_BPW = _B // _NW
_CH = 128
_NCH = _BPW // _CH

_mesh = plsc.VectorSubcoreMesh(core_axis_name="c", subcore_axis_name="s")


@functools.partial(
    pl.kernel,
    out_type=[
        jax.ShapeDtypeStruct((_B, _E), jnp.float32),
        jax.ShapeDtypeStruct((_B, _E), jnp.float32),
    ],
    mesh=_mesh,
    compiler_params=pltpu.CompilerParams(use_tc_tiling_on_sc=False),
    scratch_types=[
        pltpu.VMEM((_NCH, _CH), jnp.int32),
        pltpu.VMEM((_NCH, _CH), jnp.int32),
        pltpu.VMEM((_BPW, _E), jnp.float32),
        pltpu.VMEM((_BPW, _E), jnp.float32),
        pltpu.SemaphoreType.DMA,
        pltpu.SemaphoreType.DMA,
    ],
)
def _gather2(uidx_hbm, iidx_hbm, utab_hbm, itab_hbm, u_out, i_out,
             uidx_v, iidx_v, urows_v, irows_v, usem, isem):
    wid = lax.axis_index("s") * _NC + lax.axis_index("c")
    base = wid * _BPW
    pltpu.sync_copy(uidx_hbm.at[wid], uidx_v)
    pltpu.sync_copy(iidx_hbm.at[wid], iidx_v)
    copies = []
    for j in range(_NCH):
        copies.append(pltpu.async_copy(
            utab_hbm.at[uidx_v.at[j]], urows_v.at[pl.ds(j * _CH, _CH)], usem))
    for j in range(_NCH):
        copies.append(pltpu.async_copy(
            itab_hbm.at[iidx_v.at[j]], irows_v.at[pl.ds(j * _CH, _CH)], isem))
    for c in copies:
        c.wait()
    pltpu.sync_copy(urows_v, u_out.at[pl.ds(base, _BPW)])
    pltpu.sync_copy(irows_v, i_out.at[pl.ds(base, _BPW)])


def _mlp_body(u_ref, i_ref, w1u_ref, w1i_ref, b1_ref, g1_ref, be1_ref,
              w2_ref, b2_ref, g2_ref, be2_ref, wog_ref, woh_ref, bo_ref,
              out_ref):
    dn = (((1,), (1,)), ((), ()))
    u = u_ref[...]
    it = i_ref[...]
    h = (lax.dot_general(u, w1u_ref[...], dn, preferred_element_type=jnp.float32)
         + lax.dot_general(it, w1i_ref[...], dn, preferred_element_type=jnp.float32)
         + b1_ref[...])
    mu = jnp.mean(h, axis=0, keepdims=True)
    var = jnp.mean((h - mu) ** 2, axis=0, keepdims=True)
    h = (h - mu) * lax.rsqrt(var + 1e-5) * g1_ref[...] + be1_ref[...]
    h = jnp.maximum(h, 0.0)
    h2 = lax.dot_general(h, w2_ref[...], dn, preferred_element_type=jnp.float32) + b2_ref[...]
    mu2 = jnp.mean(h2, axis=0, keepdims=True)
    var2 = jnp.mean((h2 - mu2) ** 2, axis=0, keepdims=True)
    h2 = (h2 - mu2) * lax.rsqrt(var2 + 1e-5) * g2_ref[...] + be2_ref[...]
    h2 = jnp.maximum(h2, 0.0)
    gmf = u * it
    pred = (lax.dot_general(gmf, wog_ref[...], dn, preferred_element_type=jnp.float32)
            + lax.dot_general(h2, woh_ref[...], dn, preferred_element_type=jnp.float32)
            + bo_ref[...])
    out_ref[...] = 4.5 / (1.0 + jnp.exp(-pred)) + 0.5


_mlp = pl.pallas_call(
    _mlp_body,
    out_shape=jax.ShapeDtypeStruct((_B, 1), jnp.float32),
)


def kernel(user_indices, item_indices, user_table, item_table,
           W1, b1, g1, be1, W2, b2, g2, be2, Wo, bo):
    uidx = user_indices.astype(jnp.int32).reshape(_NW, _NCH, _CH)
    iidx = item_indices.astype(jnp.int32).reshape(_NW, _NCH, _CH)
    u, i = _gather2(uidx, iidx, user_table, item_table)
    pred = _mlp(
        u, i,
        W1[:, :_E], W1[:, _E:],
        b1.reshape(1, -1), g1.reshape(1, -1), be1.reshape(1, -1),
        W2,
        b2.reshape(1, -1), g2.reshape(1, -1), be2.reshape(1, -1),
        Wo[:, :_E], Wo[:, _E:],
        bo.reshape(1, 1),
    )
    return pred[:, 0]
